# Initial kernel scaffold; baseline (speedup 1.0000x reference)
import jax
import jax.numpy as jnp
from jax.experimental import pallas as pl


def kernel(x, W_pre, g_pre, b_pre, W_s1a, g_s1, b_s1, W_s1b, b_s1b, W_m1, g_m1, b_m1, W_s2a, g_s2, b_s2, W_s2b, b_s2b, W_m2, g_m2, b_m2, W_post, g_post, b_post):
    raise NotImplementedError("write your pallas kernel here")


# speedup vs baseline: 464.8783x; 464.8783x over previous
import functools

import jax
import jax.numpy as jnp
from jax import lax
from jax.experimental import pallas as pl
from jax.experimental.pallas import tpu as pltpu
from jax.experimental.pallas import tpu_sc as plsc

C = 64
KNN = 16
NCAND = 64
EPS = 1e-5
NW = 32
NC2 = 2

_MESH = dict(core_axis_name="c", subcore_axis_name="s")


def _wid():
    return lax.axis_index("s") * NC2 + lax.axis_index("c")


def _full16(v):
    return jnp.full((16,), v, jnp.int32)


def _tc_pre_body(x_ref, w_ref, g_ref, b_ref, h_ref):
    w = w_ref[...]
    ys = [
        lax.dot_general(x_ref[b], w, (((0,), (1,)), ((), ())),
                        preferred_element_type=jnp.float32)
        for b in range(x_ref.shape[0])
    ]
    y = jnp.concatenate(ys, axis=0)
    m = jnp.mean(y, axis=0, keepdims=True)
    v = jnp.mean((y - m) * (y - m), axis=0, keepdims=True)
    h_ref[...] = jnp.maximum((y - m) * lax.rsqrt(v + EPS) * g_ref[...]
                             + b_ref[...], 0.0)


def _tc_topk_body(d_ref, cand_ref, idx_ref):
    d = d_ref[...]
    cand = cand_ref[...]
    nt = d.shape[0]
    lane = lax.broadcasted_iota(jnp.int32, (nt, NCAND), 1)
    cols = []
    for _ in range(KNN):
        mn = jnp.min(d, axis=1, keepdims=True)
        is_mn = d == mn
        first = jnp.min(jnp.where(is_mn, lane, NCAND), axis=1, keepdims=True)
        sel = lane == first
        cols.append(jnp.sum(jnp.where(sel, cand, 0), axis=1, keepdims=True))
        d = jnp.where(sel, 1e30, d)
    idx_ref[...] = jnp.concatenate(cols, axis=1)


def _tc_stage_body(h_ref, part_ref, wa_ref, ga_ref, ba_ref, wb_ref,
                   t_ref, *, bnk):
    h = h_ref[...]
    u = lax.dot_general(h, wa_ref[...], (((1,), (1,)), ((), ())),
                        preferred_element_type=jnp.float32)
    c2 = part_ref[0].astype(jnp.float32) + part_ref[1].astype(jnp.float32)
    ccol = c2[:, 0:1]
    s1 = lax.dot_general(ccol, u, (((0,), (0,)), ((), ())),
                         preferred_element_type=jnp.float32)
    s2 = lax.dot_general(ccol, u * u, (((0,), (0,)), ((), ())),
                         preferred_element_type=jnp.float32)
    m = s1 / bnk
    v = s2 / bnk - m * m
    uh = jnp.maximum((u - m) * lax.rsqrt(v + EPS) * ga_ref[...]
                     + ba_ref[...], 0.0)
    e = jnp.exp(lax.dot_general(uh, wb_ref[...], (((1,), (1,)), ((), ())),
                                preferred_element_type=jnp.float32))
    lane = lax.broadcasted_iota(jnp.int32, e.shape, 1)
    t_ref[...] = jnp.where(lane < C, h, e)


def _tc_mid_body(p_ref, wm_ref, gm_ref, bm_ref, part_ref, wa_ref, ga_ref,
                 ba_ref, wb_ref, m_ref, *, bnk):
    p = p_ref[...]
    y = lax.dot_general(p, wm_ref[...], (((1,), (1,)), ((), ())),
                        preferred_element_type=jnp.float32)
    m = jnp.mean(y, axis=0, keepdims=True)
    v = jnp.mean((y - m) * (y - m), axis=0, keepdims=True)
    hm = jnp.maximum((y - m) * lax.rsqrt(v + EPS) * gm_ref[...]
                     + bm_ref[...], 0.0)
    u = lax.dot_general(hm, wa_ref[...], (((1,), (1,)), ((), ())),
                        preferred_element_type=jnp.float32)
    c2 = part_ref[0].astype(jnp.float32) + part_ref[1].astype(jnp.float32)
    ccol = c2[:, 0:1]
    s1 = lax.dot_general(ccol, u, (((0,), (0,)), ((), ())),
                         preferred_element_type=jnp.float32)
    s2 = lax.dot_general(ccol, u * u, (((0,), (0,)), ((), ())),
                         preferred_element_type=jnp.float32)
    mm = s1 / bnk
    vv = s2 / bnk - mm * mm
    uh = jnp.maximum((u - mm) * lax.rsqrt(vv + EPS) * ga_ref[...]
                     + ba_ref[...], 0.0)
    e = jnp.exp(lax.dot_general(uh, wb_ref[...], (((1,), (1,)), ((), ())),
                                preferred_element_type=jnp.float32))
    lane = lax.broadcasted_iota(jnp.int32, e.shape, 1)
    m_ref[...] = jnp.where(lane < C, hm, e)


def _tc_final_body(p_ref, x_ref, wm_ref, gm_ref, bm_ref, wp_ref, gp_ref,
                   bp_ref, o_ref):
    bsz, _, n = x_ref.shape
    wm = wm_ref[...]
    ys = [
        lax.dot_general(wm, p_ref[pl.ds(b * n, n)], (((1,), (1,)), ((), ())),
                        preferred_element_type=jnp.float32)
        for b in range(bsz)
    ]
    tot = bsz * n
    m = sum(jnp.sum(y, axis=1, keepdims=True) for y in ys) / tot
    v = sum(jnp.sum((y - m) * (y - m), axis=1, keepdims=True)
            for y in ys) / tot
    inv = lax.rsqrt(v + EPS)
    hs = [jnp.maximum((y - m) * inv * gm_ref[...] + bm_ref[...], 0.0)
          for y in ys]
    wp = wp_ref[...]
    zs = [lax.dot_general(wp, h, (((1,), (0,)), ((), ())),
                          preferred_element_type=jnp.float32) for h in hs]
    m2 = sum(jnp.sum(z, axis=1, keepdims=True) for z in zs) / tot
    v2 = sum(jnp.sum((z - m2) * (z - m2), axis=1, keepdims=True)
             for z in zs) / tot
    inv2 = lax.rsqrt(v2 + EPS)
    for b in range(bsz):
        z = jnp.maximum((zs[b] - m2) * inv2 * gp_ref[...] + bp_ref[...], 0.0)
        z = z + x_ref[b]
        o_ref[b] = jnp.where(z >= 0, z, 0.2 * z)


def _tc_call(body, out_shape, *args):
    return pl.pallas_call(
        body, out_shape=out_shape,
        compiler_params=pltpu.CompilerParams(
            vmem_limit_bytes=100 * 1024 * 1024))(*args)


def _sc_dist_kernel(h_hbm, cand_hbm, dist_hbm, feat_v, own_v, cidx_v, dist_v,
                    sem0, sem1, *, nt):
    ppw = nt // NW
    nblk = ppw // 32
    wid = _wid()
    lane = lax.iota(jnp.int32, 16)
    sems = (sem0, sem1)

    def gather(slot, lg):
        return pltpu.make_async_copy(h_hbm.at[cidx_v.at[lg]],
                                     feat_v.at[slot], sems[slot])

    def compute(slot, lg):
        for pp in range(2):
            lp = 2 * lg + pp
            own = [own_v[lp, pl.ds(g4 * 16, 16)] for g4 in range(4)]
            for cg in range(4):

                def cbody(c16, dv, cg=cg):
                    crow = pp * 64 + cg * 16 + c16
                    q = []
                    for g4 in range(4):
                        dlt = feat_v[slot, crow, pl.ds(g4 * 16, 16)] - own[g4]
                        q.append(dlt * dlt)
                    dsc = jnp.sum((q[0] + q[1]) + (q[2] + q[3]))
                    return jnp.where(lane == c16, dsc, dv)

                dist_v[lp, pl.ds(cg * 16, 16)] = lax.fori_loop(
                    0, 16, cbody, jnp.zeros((16,), jnp.float32))

    def blk_body(bi, _):
        pbase = pl.multiple_of(wid * ppw + bi * 32, 32)
        gbase = pl.multiple_of(pbase // 2, 16)
        pltpu.sync_copy(h_hbm.at[pl.ds(pbase, 32)], own_v)
        pltpu.sync_copy(cand_hbm.at[pl.ds(gbase, 16)], cidx_v)
        gather(0, 0).start()

        def pair(i, _):
            lg0 = 2 * i
            gather(1, lg0 + 1).start()
            gather(0, lg0).wait()
            compute(0, lg0)
            gather(0, jnp.minimum(lg0 + 2, 15)).start()
            gather(1, lg0 + 1).wait()
            compute(1, lg0 + 1)
            return 0

        lax.fori_loop(0, 8, pair, 0)
        gather(0, 15).wait()
        pltpu.sync_copy(dist_v, dist_hbm.at[pl.ds(pbase, 32)])
        return 0

    lax.fori_loop(0, nblk, blk_body, 0)


def _sc_count_kernel(idx_hbm, cnt_hbm, shared, idxb, tidx, fill_v, *, nt):
    sid = lax.axis_index("s")
    cid = lax.axis_index("c")
    wid = sid * NC2 + cid
    qtr = nt // 4
    srows = 4224
    rpw = (nt * KNN // 128) // NW

    def fill(hi, val):
        def frow(i, _):
            for j in range(8):
                fill_v[i, pl.ds(16 * j, 16)] = jnp.full((16,), val,
                                                        jnp.float32)
            return 0

        lax.fori_loop(0, hi, frow, 0)

    pltpu.sync_copy(idx_hbm.at[pl.ds(pl.multiple_of(wid * rpw, 64), rpw)],
                    idxb)
    for ph in range(4):
        lo = ph * qtr
        fill(264, 0.0)
        pltpu.sync_copy(fill_v.at[pl.ds(0, 264)],
                        shared.at[pl.ds(pl.multiple_of(sid * 264, 8), 264)])
        plsc.subcore_barrier()
        fill(128, 1.0)

        def trow(r, _):
            for c8 in range(8):
                v = idxb[r, pl.ds(c8 * 16, 16)]
                inr = (v >= lo) & (v < lo + qtr)
                tv = jnp.where(inr, v - lo, qtr + (v & 127))
                tidx[r, pl.ds(c8 * 16, 16)] = tv
            return 0

        lax.fori_loop(0, rpw, trow, 0)

        def scat(j, _):
            pltpu.sync_copy(fill_v.at[pl.ds(0, 128)], shared.at[tidx.at[j]],
                            add=True)
            return 0

        lax.fori_loop(0, rpw, scat, 0)
        plsc.subcore_barrier()
        srow = pl.multiple_of(sid * 256, 8)
        pltpu.sync_copy(shared.at[pl.ds(srow, 256)],
                        cnt_hbm.at[cid, pl.ds(lo + srow, 256)])
        plsc.subcore_barrier()


def _sc_pool_kernel(h_hbm, idx_hbm, out_hbm, feat_v, idxb_v, out_v,
                    sem0, sem1, *, nt):
    ppw = nt // NW
    nblk = ppw // 128
    wid = _wid()
    sems = (sem0, sem1)

    def gather(slot, lg):
        return pltpu.make_async_copy(h_hbm.at[idxb_v.at[lg]],
                                     feat_v.at[slot], sems[slot])

    def compute(slot, lg):
        for pp in range(8):
            es = [feat_v[slot, pp * 16 + k, pl.ds(C, 16)] for k in range(KNN)]
            tot = es[0]
            for k in range(1, KNN):
                tot = tot + es[k]
            inv = 1.0 / tot
            acc = [jnp.zeros((16,), jnp.float32) for _ in range(4)]
            for k in range(KNN):
                wk = es[k] * inv
                for g4 in range(4):
                    acc[g4] = acc[g4] + wk * feat_v[slot, pp * 16 + k,
                                                    pl.ds(g4 * 16, 16)]
            for g4 in range(4):
                out_v[lg * 8 + pp, pl.ds(g4 * 16, 16)] = acc[g4]

    def blk_body(bi, _):
        pbase = pl.multiple_of(wid * ppw + bi * 128, 128)
        gbase = pl.multiple_of(pbase // 8, 16)
        pltpu.sync_copy(idx_hbm.at[pl.ds(gbase, 16)], idxb_v)
        gather(0, 0).start()

        def pair(i, _):
            lg0 = 2 * i
            gather(1, lg0 + 1).start()
            gather(0, lg0).wait()
            compute(0, lg0)
            gather(0, jnp.minimum(lg0 + 2, 15)).start()
            gather(1, lg0 + 1).wait()
            compute(1, lg0 + 1)
            return 0

        lax.fori_loop(0, 8, pair, 0)
        gather(0, 15).wait()
        pltpu.sync_copy(out_v, out_hbm.at[pl.ds(pbase, 128)])
        return 0

    lax.fori_loop(0, nblk, blk_body, 0)


def _sc_dist(h_rows, cand3d, nt):
    mesh = plsc.VectorSubcoreMesh(**_MESH)
    fn = functools.partial(
        pl.kernel, mesh=mesh,
        compiler_params=pltpu.CompilerParams(needs_layout_passes=False),
        out_type=jax.ShapeDtypeStruct((nt, NCAND), jnp.float32),
        scratch_types=[
            pltpu.VMEM((2, 128, 2 * C), jnp.float32),
            pltpu.VMEM((32, 2 * C), jnp.float32),
            pltpu.VMEM((16, 128), jnp.int32),
            pltpu.VMEM((32, NCAND), jnp.float32),
            pltpu.SemaphoreType.DMA,
            pltpu.SemaphoreType.DMA,
        ])(functools.partial(_sc_dist_kernel, nt=nt))
    return fn(h_rows, cand3d)


def _sc_counts(idx3d, nt):
    mesh = plsc.VectorSubcoreMesh(**_MESH)
    fn = functools.partial(
        pl.kernel, mesh=mesh,
        compiler_params=pltpu.CompilerParams(needs_layout_passes=False),
        out_type=jax.ShapeDtypeStruct((2, nt, 128), jnp.float32),
        scratch_types=[
            pltpu.VMEM_SHARED((4224, 128), jnp.float32),
            pltpu.VMEM((64, 128), jnp.int32),
            pltpu.VMEM((64, 128), jnp.int32),
            pltpu.VMEM((264, 128), jnp.float32),
        ])(functools.partial(_sc_count_kernel, nt=nt))
    return fn(idx3d)


def _sc_pool(h_rows, idx3d, nt):
    mesh = plsc.VectorSubcoreMesh(**_MESH)
    fn = functools.partial(
        pl.kernel, mesh=mesh,
        compiler_params=pltpu.CompilerParams(needs_layout_passes=False),
        out_type=jax.ShapeDtypeStruct((nt, C), jnp.float32),
        scratch_types=[
            pltpu.VMEM((2, 128, 2 * C), jnp.float32),
            pltpu.VMEM((16, 128), jnp.int32),
            pltpu.VMEM((128, C), jnp.float32),
            pltpu.SemaphoreType.DMA,
            pltpu.SemaphoreType.DMA,
        ])(functools.partial(_sc_pool_kernel, nt=nt))
    return fn(h_rows, idx3d)


def kernel(x, W_pre, g_pre, b_pre, W_s1a, g_s1, b_s1, W_s1b, b_s1b,
           W_m1, g_m1, b_m1, W_s2a, g_s2, b_s2, W_s2b, b_s2b,
           W_m2, g_m2, b_m2, W_post, g_post, b_post):
    B, Cin, N = x.shape
    nt = B * N
    bnk = float(nt * KNN)
    row = lambda a: a.reshape(1, -1)
    col = lambda a: a.reshape(-1, 1)
    padw = lambda w: jnp.concatenate([w, jnp.zeros_like(w)], axis=0)
    padv = lambda a: jnp.concatenate([row(a), jnp.zeros((1, C), a.dtype)],
                                     axis=1)

    h1 = _tc_call(_tc_pre_body,
                  jax.ShapeDtypeStruct((nt, 2 * C), jnp.float32),
                  x, padw(W_pre), padv(g_pre), padv(b_pre))

    cand = jax.random.randint(jax.random.key(42), (B, N, NCAND), 0, N)
    cand_flat = (cand.astype(jnp.int32)
                 + (jnp.arange(B, dtype=jnp.int32) * N)[:, None, None]
                 ).reshape(nt, NCAND)
    cand3d = cand_flat.reshape(nt * NCAND // 128, 128)

    dists = _sc_dist(h1, cand3d, nt)
    idx_flat = _tc_call(_tc_topk_body,
                        jax.ShapeDtypeStruct((nt, KNN), jnp.int32),
                        dists, cand_flat)
    idx3d = idx_flat.reshape(nt * KNN // 128, 128)

    parts = _sc_counts(idx3d, nt)

    h1e = _tc_call(functools.partial(_tc_stage_body, bnk=bnk),
                   jax.ShapeDtypeStruct((nt, 2 * C), jnp.float32),
                   h1, parts, padw(W_s1a.T).T, row(g_s1), row(b_s1),
                   jnp.tile(W_s1b, (2 * C, 1)))
    pooled1 = _sc_pool(h1e, idx3d, nt)

    m1e = _tc_call(functools.partial(_tc_mid_body, bnk=bnk),
                   jax.ShapeDtypeStruct((nt, 2 * C), jnp.float32),
                   pooled1, padw(W_m1), padv(g_m1), padv(b_m1), parts,
                   padw(W_s2a.T).T, row(g_s2), row(b_s2),
                   jnp.tile(W_s2b, (2 * C, 1)))
    pooled2 = _sc_pool(m1e, idx3d, nt)

    out = _tc_call(_tc_final_body,
                   jax.ShapeDtypeStruct((B, Cin, N), jnp.float32),
                   pooled2, x, W_m2, col(g_m2), col(b_m2),
                   W_post, col(g_post), col(b_post))
    return out
